# baseline (speedup 1.0000x reference)
import functools

import jax
import jax.numpy as jnp
from jax import lax
from jax.experimental import pallas as pl
from jax.experimental.pallas import tpu as pltpu
from jax.experimental.pallas import tpu_sc as plsc

NUM_CORES = 2
NUM_SUBCORES = 16
NUM_WORKERS = NUM_CORES * NUM_SUBCORES

BATCH = 16384
DIM = 32
NPARTS = 1000000
LANES = 16

PANEL_W = 2048
N_UNITS = 489
TAIL_W = 640
CAP = 1024

_mesh = plsc.VectorSubcoreMesh(core_axis_name="c", subcore_axis_name="s")


@functools.partial(
    pl.kernel,
    mesh=_mesh,
    out_type=jax.ShapeDtypeStruct((BATCH + 1, 128), jnp.float32),
    compiler_params=pltpu.CompilerParams(needs_layout_passes=False),
    scratch_types=[
        pltpu.VMEM((BATCH,), jnp.int32),
        pltpu.VMEM((CAP,), jnp.int32),
        pltpu.VMEM((CAP,), jnp.int32),
        pltpu.VMEM((128,), jnp.int32),
        pltpu.VMEM((128,), jnp.int32),
        pltpu.VMEM((DIM, PANEL_W), jnp.float32),
        pltpu.VMEM((LANES, 128), jnp.float32),
        pltpu.VMEM((LANES, 128), jnp.float32),
        pltpu.SemaphoreType.DMA,
        pltpu.SemaphoreType.DMA,
    ],
)
def _sweep(idx_hbm, table_t_hbm, out_hbm, idx_v, t_v, pos_v, pt_v, ppos_v,
           panel_v, buf_a, buf_b, sem_a, sem_b):
    wid = lax.axis_index("s") * NUM_CORES + lax.axis_index("c")
    pltpu.sync_copy(idx_hbm, idx_v)

    iota16 = lax.iota(jnp.int32, LANES)
    rows_lo = iota16
    rows_hi = iota16 + LANES

    def filt(k, m):
        tvec = idx_v[pl.ds(k * LANES, LANES)]
        mine = lax.bitwise_and(lax.shift_right_logical(tvec, 11), 31) == wid
        sel = jnp.where(mine, 1, 0).astype(jnp.int32)
        csum = plsc.cumsum(sel)
        dest = m + csum - 1
        plsc.store_scatter(t_v, [dest], tvec, mask=mine)
        plsc.store_scatter(pos_v, [dest], iota16 + k * LANES, mask=mine)
        return m + csum[LANES - 1]

    m_cnt = lax.fori_loop(0, BATCH // LANES, filt, jnp.int32(0))

    n_units_mine = (N_UNITS - 1 - wid) // 32 + 1

    def unit_body(j, n_sc):
        u = wid + 32 * j
        start = u * PANEL_W
        is_tail = u == N_UNITS - 1
        width = jnp.where(is_tail, TAIL_W, PANEL_W)

        @pl.when(jnp.logical_not(is_tail))
        def _():
            pltpu.sync_copy(
                table_t_hbm.at[:, pl.ds(pl.multiple_of(start, 128), PANEL_W)],
                panel_v,
            )

        @pl.when(is_tail)
        def _():
            pltpu.sync_copy(
                table_t_hbm.at[:, pl.ds(pl.multiple_of(start, 128), TAIL_W)],
                panel_v.at[:, pl.ds(0, TAIL_W)],
            )

        def refilt(q, mp):
            lane_pos = iota16 + q * LANES
            tq = t_v[pl.ds(q * LANES, LANES)]
            pq = pos_v[pl.ds(q * LANES, LANES)]
            inu = (
                (lane_pos < m_cnt) & (tq >= start) & (tq < start + width)
            )
            sel = jnp.where(inu, 1, 0).astype(jnp.int32)
            csum = plsc.cumsum(sel)
            dest = mp + csum - 1
            plsc.store_scatter(pt_v, [dest], tq - start, mask=inu)
            plsc.store_scatter(ppos_v, [dest], pq, mask=inu)
            return mp + csum[LANES - 1]

        mp_cnt = lax.fori_loop(
            0, (m_cnt + LANES - 1) // LANES, refilt, jnp.int32(0)
        )

        def block_body(r, n_sc_in):
            lane_pos = iota16 + r * LANES
            valid = lane_pos < mp_cnt
            cols = jnp.where(valid, pt_v[pl.ds(r * LANES, LANES)], 0)
            rows_out = jnp.where(
                valid, ppos_v[pl.ds(r * LANES, LANES)], BATCH
            )

            def emit(buf, sem, fired_before):
                @pl.when(fired_before)
                def _():
                    pltpu.make_async_copy(
                        buf, out_hbm.at[rows_out], sem
                    ).wait()
                for k in range(LANES):
                    ck = jnp.full((LANES,), cols[k], jnp.int32)
                    v_lo = plsc.load_gather(panel_v, [rows_lo, ck])
                    v_hi = plsc.load_gather(panel_v, [rows_hi, ck])
                    buf[k, pl.ds(0, LANES)] = v_lo
                    buf[k, pl.ds(LANES, LANES)] = v_hi
                pltpu.async_copy(buf, out_hbm.at[rows_out], sem)

            parity = lax.bitwise_and(n_sc_in, 1)

            @pl.when(parity == 0)
            def _():
                emit(buf_a, sem_a, n_sc_in >= 2)

            @pl.when(parity == 1)
            def _():
                emit(buf_b, sem_b, n_sc_in >= 2)

            return n_sc_in + 1

        return lax.fori_loop(
            0, (mp_cnt + LANES - 1) // LANES, block_body, n_sc
        )

    n_sc = lax.fori_loop(0, n_units_mine, unit_body, jnp.int32(0))

    dummy = jnp.full((LANES,), BATCH, jnp.int32)

    @pl.when(n_sc >= 1)
    def _():
        pltpu.make_async_copy(buf_a, out_hbm.at[dummy], sem_a).wait()

    @pl.when(n_sc >= 2)
    def _():
        pltpu.make_async_copy(buf_b, out_hbm.at[dummy], sem_b).wait()


def kernel(indices, latents):
    inter = _sweep(indices.astype(jnp.int32), latents.T)
    return inter[:BATCH, :DIM]


# speedup vs baseline: 2.1302x; 2.1302x over previous
import functools

import jax
import jax.numpy as jnp
from jax import lax
from jax.experimental import pallas as pl
from jax.experimental.pallas import tpu as pltpu
from jax.experimental.pallas import tpu_sc as plsc

NUM_CORES = 2
NUM_SUBCORES = 16
NUM_WORKERS = NUM_CORES * NUM_SUBCORES

BATCH = 16384
DIM = 32
B_PER_W = BATCH // NUM_WORKERS
LANES = 16
BATCH_IDX = 16
N_ROUNDS = B_PER_W // BATCH_IDX

_mesh = plsc.VectorSubcoreMesh(core_axis_name="c", subcore_axis_name="s")


@functools.partial(
    pl.kernel,
    mesh=_mesh,
    out_type=jax.ShapeDtypeStruct((DIM, BATCH), jnp.float32),
    compiler_params=pltpu.CompilerParams(needs_layout_passes=False),
    scratch_types=[
        pltpu.VMEM((B_PER_W,), jnp.int32),
        pltpu.VMEM((BATCH_IDX, DIM, 128), jnp.float32),
        pltpu.VMEM((DIM, B_PER_W), jnp.float32),
        pltpu.SemaphoreType.DMA,
    ],
)
def _gather_t(idx_hbm, table_t_hbm, out_t_hbm, idx_v, chunks_v, cols_v, sem):
    wid = lax.axis_index("s") * NUM_CORES + lax.axis_index("c")
    base = wid * B_PER_W
    pltpu.sync_copy(idx_hbm.at[pl.ds(base, B_PER_W)], idx_v)

    rows_lo = lax.iota(jnp.int32, LANES)
    rows_hi = rows_lo + LANES

    def round_body(b, carry):
        tvec = idx_v[pl.ds(b * BATCH_IDX, BATCH_IDX)]
        g_vec = lax.shift_right_logical(tvec, 7)
        c_vec = lax.bitwise_and(tvec, 127)
        for l in range(BATCH_IDX):
            start = pl.multiple_of(g_vec[l] * 128, 128)
            pltpu.async_copy(
                table_t_hbm.at[:, pl.ds(start, 128)], chunks_v.at[l], sem
            )
        for l in range(BATCH_IDX):
            pltpu.make_async_copy(
                table_t_hbm.at[:, pl.ds(0, 128)], chunks_v.at[l], sem
            ).wait()
        for l in range(BATCH_IDX):
            col = jnp.full((LANES,), c_vec[l], jnp.int32)
            i_col = jnp.full((LANES,), b * BATCH_IDX + l, jnp.int32)
            v_lo = plsc.load_gather(chunks_v.at[l], [rows_lo, col])
            v_hi = plsc.load_gather(chunks_v.at[l], [rows_hi, col])
            plsc.store_scatter(cols_v, [rows_lo, i_col], v_lo)
            plsc.store_scatter(cols_v, [rows_hi, i_col], v_hi)
        return carry

    lax.fori_loop(0, N_ROUNDS, round_body, 0)
    pltpu.sync_copy(cols_v, out_t_hbm.at[:, pl.ds(base, B_PER_W)])


def kernel(indices, latents):
    out_t = _gather_t(indices.astype(jnp.int32), latents.T)
    return out_t.T
